# baseline (speedup 1.0000x reference)
import jax
import jax.numpy as jnp
from jax import lax
from jax.experimental import pallas as pl
from jax.experimental.pallas import tpu as pltpu
from jax.experimental.pallas import tpu_sc as plsc

N = 10000
E = 160000
IN_CH = 256
HID = 256
OUT2 = 64
NUM_CLASSES = 10
NUM_GRAPHS = 8

L = 16
NC, NS = 2, 16
MW = 128
E_PAD = 161280


def _sc_edge_kernel(mreal, feature_split, with_cnt, chunk):
  R = 2 * mreal
  ept = (E_PAD if feature_split else E_PAD // NC) // NS
  nchunks = ept // chunk
  assert nchunks * chunk == ept and chunk % 16 == 0 and chunk <= 128
  mesh = plsc.VectorSubcoreMesh(core_axis_name="c", subcore_axis_name="s",
                                num_cores=NC, num_subcores=NS)
  if with_cnt:
    out_type = (jax.ShapeDtypeStruct((NC, N, MW), jnp.float32),
                jax.ShapeDtypeStruct((N,), jnp.float32))
  else:
    out_type = jax.ShapeDtypeStruct((NC, N, MW), jnp.float32)
  scratch = [
      pltpu.VMEM_SHARED((N, MW), jnp.float32),
      pltpu.VMEM_SHARED((N,), jnp.float32),
      pltpu.VMEM((N,), jnp.float32),
      pltpu.VMEM((L,), jnp.float32),
      [pltpu.VMEM((chunk,), jnp.int32)] * 2,
      [pltpu.VMEM((chunk,), jnp.int32)] * 2,
      [pltpu.VMEM((chunk,), jnp.float32)] * 2,
      [pltpu.VMEM((chunk,), jnp.float32)] * 2,
      [pltpu.VMEM((chunk,), jnp.float32)] * 2,
      [pltpu.VMEM((chunk, R), jnp.float32)] * 2,
      pltpu.VMEM((chunk, MW), jnp.float32),
      pltpu.VMEM((16, MW), jnp.float32),
      pltpu.VMEM((640,), jnp.float32),
      [pltpu.SemaphoreType.DMA] * 2,
      [pltpu.SemaphoreType.DMA] * 2,
  ]

  def body(ta_hbm, tb_hbm, y_hbm, dc_hbm, src_hbm, dst_hbm, agg_out, *rest):
    if with_cnt:
      cnt_out = rest[0]
      rest = rest[1:]
    (agg_s, cnt_s, y_v, dc_v, src_c, dst_c, q0_v, q1_v, val_v,
     rows_v, msg_v, zblk, zcnt, isem, gsem) = rest
    c = lax.axis_index("c")
    s = lax.axis_index("s")
    zero = jnp.zeros((L,), jnp.float32)

    pltpu.sync_copy(y_hbm, y_v)
    pltpu.sync_copy(dc_hbm, dc_v)

    start = jnp.minimum(s * 640, N - 640)

    def zrow(r, _):
      for ch in range(MW // L):
        zblk[r, pl.ds(ch * L, L)] = zero
      return 0
    lax.fori_loop(0, 16, zrow, 0)
    for g in range(640 // L):
      zcnt[pl.ds(g * L, L)] = zero
    for e in range(chunk):
      for ch in range(mreal // L, MW // L):
        msg_v[e, pl.ds(ch * L, L)] = zero

    def zcopy(j, _):
      pltpu.sync_copy(zblk, agg_s.at[pl.ds(start + j * 16, 16)])
      return 0
    lax.fori_loop(0, 640 // 16, zcopy, 0)
    if with_cnt:
      pltpu.sync_copy(zcnt, cnt_s.at[pl.ds(start, 640)])
    plsc.subcore_barrier()

    ebase = s * ept if feature_split else (c * NS + s) * ept
    dcb = dc_v[...]

    def issue_idx(k, b):
      base = ebase + k * chunk
      pltpu.make_async_copy(src_hbm.at[pl.ds(base, chunk)], src_c[b],
                            isem[b]).start()
      pltpu.make_async_copy(dst_hbm.at[pl.ds(base, chunk)], dst_c[b],
                            isem[b]).start()

    def wait_idx(k, b):
      base = ebase + k * chunk
      pltpu.make_async_copy(src_hbm.at[pl.ds(base, chunk)], src_c[b],
                            isem[b]).wait()
      pltpu.make_async_copy(dst_hbm.at[pl.ds(base, chunk)], dst_c[b],
                            isem[b]).wait()

    def qcomp(b):
      for g in range(chunk // L):
        sv = src_c[b][pl.ds(g * L, L)]
        dv = dst_c[b][pl.ds(g * L, L)]
        yj = plsc.load_gather(y_v, [sv])
        yi = plsc.load_gather(y_v, [dv])
        q0 = 1.0 / (1.0 + jnp.exp(yi - yj - dcb))
        validf = jnp.where(sv != dv, 1.0, 0.0).astype(jnp.float32)
        q0 = q0 * validf
        q0_v[b][pl.ds(g * L, L)] = q0
        q1_v[b][pl.ds(g * L, L)] = validf - q0
        val_v[b][pl.ds(g * L, L)] = validf

    def issue_gather(b):
      @pl.when(c == 0)
      def _():
        pltpu.make_async_copy(ta_hbm.at[src_c[b]], rows_v[b],
                              gsem[b]).start()
      @pl.when(c == 1)
      def _():
        pltpu.make_async_copy(tb_hbm.at[src_c[b]], rows_v[b],
                              gsem[b]).start()

    def wait_gather(b):
      pltpu.make_async_copy(ta_hbm.at[src_c[b]], rows_v[b], gsem[b]).wait()

    def combine_scatter(b):
      def comb(e, _):
        eix = jnp.zeros((L,), jnp.int32) + e
        q0b = plsc.load_gather(q0_v[b], [eix])
        q1b = plsc.load_gather(q1_v[b], [eix])
        for ch in range(mreal // L):
          r0 = rows_v[b][e, pl.ds(ch * L, L)]
          r1 = rows_v[b][e, pl.ds(mreal + ch * L, L)]
          msg_v[e, pl.ds(ch * L, L)] = q0b * r0 + q1b * r1
        return 0
      lax.fori_loop(0, chunk, comb, 0)
      pltpu.sync_copy(msg_v, agg_s.at[dst_c[b]], add=True)
      if with_cnt:
        @pl.when(c == 0)
        def _():
          pltpu.sync_copy(val_v[b], cnt_s.at[dst_c[b]], add=True)

    n_pipe = nchunks - 2 if nchunks % 2 == 0 else nchunks - 3

    issue_idx(0, 0)
    issue_idx(1, 1)
    wait_idx(0, 0)
    qcomp(0)
    issue_gather(0)

    def pipe_pair(j, _):
      k = 2 * j
      for b in (0, 1):
        kk = k + b
        wait_idx(kk + 1, 1 - b)
        qcomp(1 - b)
        issue_gather(1 - b)
        wait_gather(b)
        combine_scatter(b)
        issue_idx(kk + 2, b)
      return 0

    lax.fori_loop(0, n_pipe // 2, pipe_pair, 0)

    for k in range(n_pipe, nchunks):
      b = k & 1
      if k + 1 < nchunks:
        wait_idx(k + 1, 1 - b)
        qcomp(1 - b)
        issue_gather(1 - b)
      wait_gather(b)
      combine_scatter(b)
      if k + 2 < nchunks:
        issue_idx(k + 2, b)

    plsc.subcore_barrier()

    def wout(j, _):
      pltpu.sync_copy(agg_s.at[pl.ds(start + j * 32, 32)],
                      msg_v.at[pl.ds(0, 32)])
      pltpu.sync_copy(msg_v.at[pl.ds(0, 32)],
                      agg_out.at[c, pl.ds(start + j * 32, 32)])
      return 0
    lax.fori_loop(0, 640 // 32, wout, 0)
    if with_cnt:
      @pl.when(c == 0)
      def _():
        pltpu.sync_copy(cnt_s.at[pl.ds(start, 640)], zcnt)
        pltpu.sync_copy(zcnt, cnt_out.at[pl.ds(start, 640)])

  return pl.kernel(
      body, out_type=out_type, mesh=mesh, scratch_types=scratch,
      compiler_params=pltpu.CompilerParams(needs_layout_passes=False,
                                           use_tc_tiling_on_sc=False))


_sc_layer1 = _sc_edge_kernel(128, feature_split=True, with_cnt=True, chunk=48)
_sc_layer2 = _sc_edge_kernel(64, feature_split=False, with_cnt=False,
                             chunk=48)


_BLK = 1000


def _k1_body(x_ref, w_ref, du_ref, xwa_ref, xwb_ref, ys_ref):
  xb = x_ref[...]
  xw = jnp.dot(xb, w_ref[...], preferred_element_type=jnp.float32)
  xwa_ref[...] = xw[:, :HID]
  xwb_ref[...] = xw[:, HID:]
  ys_ref[...] = jnp.dot(xb, du_ref[...], preferred_element_type=jnp.float32)


def _k1_call(x, w1r, du):
  grid = N // _BLK
  return pl.pallas_call(
      _k1_body,
      grid=(grid,),
      in_specs=[
          pl.BlockSpec((_BLK, IN_CH), lambda i: (i, 0)),
          pl.BlockSpec((IN_CH, 2 * HID), lambda i: (0, 0)),
          pl.BlockSpec((IN_CH, 1), lambda i: (0, 0)),
      ],
      out_specs=[
          pl.BlockSpec((_BLK, HID), lambda i: (i, 0)),
          pl.BlockSpec((_BLK, HID), lambda i: (i, 0)),
          pl.BlockSpec((_BLK, 1), lambda i: (i, 0)),
      ],
      out_shape=[
          jax.ShapeDtypeStruct((N, HID), jnp.float32),
          jax.ShapeDtypeStruct((N, HID), jnp.float32),
          jax.ShapeDtypeStruct((N, 1), jnp.float32),
      ],
  )(x, w1r, du)


def _norm_relu(agg, xw, cnt, brow, qc):
  M = agg.shape[-1]
  self_msg = qc[0, 0] * xw[:, :M] + qc[0, 1] * xw[:, M:]
  return jnp.maximum((agg + self_msg) / (cnt + 1.0) + brow, 0.0)


def _k2_body(agg_ref, xwa_ref, xwb_ref, cnt_ref, b_ref, c1_ref, w2_ref,
             du_ref, xw2_ref, ys2_ref):
  qc = jax.nn.softmax(c1_ref[...], axis=1)
  cnt = cnt_ref[...]
  b = b_ref[...]
  h_lo = _norm_relu(agg_ref[0], xwa_ref[...], cnt, b[0:1, :], qc)
  h_hi = _norm_relu(agg_ref[1], xwb_ref[...], cnt, b[1:2, :], qc)
  hb = jnp.concatenate([h_lo, h_hi], axis=1)
  xw2_ref[...] = jnp.dot(hb, w2_ref[...], preferred_element_type=jnp.float32)
  ys2_ref[...] = jnp.dot(hb, du_ref[...], preferred_element_type=jnp.float32)


def _k2_call(agg1, xwa, xwb, cnt, b1, c1, w2, du2):
  grid = N // _BLK
  return pl.pallas_call(
      _k2_body,
      grid=(grid,),
      in_specs=[
          pl.BlockSpec((NC, _BLK, HID // 2), lambda i: (0, i, 0)),
          pl.BlockSpec((_BLK, HID), lambda i: (i, 0)),
          pl.BlockSpec((_BLK, HID), lambda i: (i, 0)),
          pl.BlockSpec((_BLK, 1), lambda i: (i, 0)),
          pl.BlockSpec((2, HID // 2), lambda i: (0, 0)),
          pl.BlockSpec((1, 2), lambda i: (0, 0)),
          pl.BlockSpec((HID, 2 * OUT2), lambda i: (0, 0)),
          pl.BlockSpec((HID, 1), lambda i: (0, 0)),
      ],
      out_specs=[
          pl.BlockSpec((_BLK, 2 * OUT2), lambda i: (i, 0)),
          pl.BlockSpec((_BLK, 1), lambda i: (i, 0)),
      ],
      out_shape=[
          jax.ShapeDtypeStruct((N, 2 * OUT2), jnp.float32),
          jax.ShapeDtypeStruct((N, 1), jnp.float32),
      ],
  )(agg1, xwa, xwb, cnt, b1, c1, w2, du2)


def _k3_body(agg2_ref, xw2_ref, cnt_ref, b_ref, c2_ref, batch_ref, wfc_ref,
             bfc_ref, out_ref):
  qc = jax.nn.softmax(c2_ref[...], axis=1)
  cnt = cnt_ref[...]
  xw2 = xw2_ref[...]
  agg2 = agg2_ref[0][:, :OUT2] + agg2_ref[1][:, :OUT2]
  self_msg = qc[0, 0] * xw2[:, :OUT2] + qc[0, 1] * xw2[:, OUT2:]
  h2 = jnp.maximum((agg2 + self_msg) / (cnt + 1.0) + b_ref[...], 0.0)
  gids = lax.broadcasted_iota(jnp.int32, (1, NUM_GRAPHS), 1)
  seg = (batch_ref[...] == gids).astype(jnp.float32)
  gsum = lax.dot_general(seg, h2, (((0,), (0,)), ((), ())),
                         preferred_element_type=jnp.float32)
  counts = lax.dot_general(seg, jnp.ones((N, 1), jnp.float32),
                           (((0,), (0,)), ((), ())),
                           preferred_element_type=jnp.float32)
  g = gsum / jnp.maximum(counts, 1.0)
  out_ref[...] = jnp.dot(g, wfc_ref[...],
                         preferred_element_type=jnp.float32) + bfc_ref[...]


def _k3_call(agg2, xw2, cnt, b2, c2, batch2d, wfc, bfc):
  return pl.pallas_call(
      _k3_body,
      out_shape=jax.ShapeDtypeStruct((NUM_GRAPHS, NUM_CLASSES), jnp.float32),
  )(agg2, xw2, cnt, b2, c2, batch2d, wfc, bfc)


def kernel(x, edge_index, batch, W1, u1, c1, b1, W2, u2, c2, b2, Wfc, bfc):
  pad = jnp.zeros((E_PAD - E,), jnp.int32)
  src = jnp.concatenate([edge_index[0], pad])
  dst = jnp.concatenate([edge_index[1], pad])
  H2 = HID // 2
  W1r = jnp.concatenate([W1[:, 0:H2], W1[:, HID:HID + H2],
                         W1[:, H2:HID], W1[:, HID + H2:]], axis=1)
  du1 = (u1[:, 0] - u1[:, 1])[:, None]
  dc1 = jnp.full((L,), c1[0] - c1[1], jnp.float32)
  du2 = (u2[:, 0] - u2[:, 1])[:, None]
  dc2 = jnp.full((L,), c2[0] - c2[1], jnp.float32)

  xwa, xwb, ys1 = _k1_call(x, W1r, du1)
  agg1, cnt = _sc_layer1(xwa, xwb, jnp.ravel(ys1), dc1, src, dst)

  xw2, ys2 = _k2_call(agg1, xwa, xwb, cnt.reshape(N, 1), b1.reshape(2, H2),
                      c1.reshape(1, 2), W2, du2)
  agg2 = _sc_layer2(xw2, xw2, jnp.ravel(ys2), dc2, src, dst)

  return _k3_call(agg2, xw2, cnt.reshape(N, 1), b2.reshape(1, OUT2),
                  c2.reshape(1, 2), batch.reshape(N, 1), Wfc, bfc[None, :])


# speedup vs baseline: 1.0153x; 1.0153x over previous
import jax
import jax.numpy as jnp
from jax import lax
from jax.experimental import pallas as pl
from jax.experimental.pallas import tpu as pltpu
from jax.experimental.pallas import tpu_sc as plsc

N = 10000
E = 160000
IN_CH = 256
HID = 256
OUT2 = 64
NUM_CLASSES = 10
NUM_GRAPHS = 8

L = 16
NC, NS = 2, 16
MW = 128
E_PAD = 161280


def _sc_edge_kernel(mreal, feature_split, with_cnt, chunk):
  R = 2 * mreal
  ept = (E_PAD if feature_split else E_PAD // NC) // NS
  nchunks = ept // chunk
  assert nchunks * chunk == ept and chunk % 16 == 0 and chunk <= 128
  mesh = plsc.VectorSubcoreMesh(core_axis_name="c", subcore_axis_name="s",
                                num_cores=NC, num_subcores=NS)
  if with_cnt:
    out_type = (jax.ShapeDtypeStruct((NC, N, MW), jnp.float32),
                jax.ShapeDtypeStruct((N,), jnp.float32))
  else:
    out_type = jax.ShapeDtypeStruct((NC, N, MW), jnp.float32)
  scratch = [
      pltpu.VMEM_SHARED((N, MW), jnp.float32),
      pltpu.VMEM_SHARED((N,), jnp.float32),
      pltpu.VMEM((N,), jnp.float32),
      pltpu.VMEM((L,), jnp.float32),
      [pltpu.VMEM((chunk,), jnp.int32)] * 2,
      [pltpu.VMEM((chunk,), jnp.int32)] * 2,
      [pltpu.VMEM((chunk,), jnp.float32)] * 2,
      [pltpu.VMEM((chunk,), jnp.float32)] * 2,
      [pltpu.VMEM((chunk,), jnp.float32)] * 2,
      [pltpu.VMEM((chunk, R), jnp.float32)] * 2,
      pltpu.VMEM((chunk, MW), jnp.float32),
      pltpu.VMEM((16, MW), jnp.float32),
      pltpu.VMEM((640,), jnp.float32),
      [pltpu.SemaphoreType.DMA] * 2,
      [pltpu.SemaphoreType.DMA] * 2,
  ]

  def body(ta_hbm, tb_hbm, y_hbm, dc_hbm, src_hbm, dst_hbm, agg_out, *rest):
    if with_cnt:
      cnt_out = rest[0]
      rest = rest[1:]
    (agg_s, cnt_s, y_v, dc_v, src_c, dst_c, q0_v, q1_v, val_v,
     rows_v, msg_v, zblk, zcnt, isem, gsem) = rest
    c = lax.axis_index("c")
    s = lax.axis_index("s")
    zero = jnp.zeros((L,), jnp.float32)

    pltpu.sync_copy(y_hbm, y_v)
    pltpu.sync_copy(dc_hbm, dc_v)

    start = jnp.minimum(s * 640, N - 640)

    def zrow(r, _):
      for ch in range(MW // L):
        zblk[r, pl.ds(ch * L, L)] = zero
      return 0
    lax.fori_loop(0, 16, zrow, 0)
    for g in range(640 // L):
      zcnt[pl.ds(g * L, L)] = zero
    for e in range(chunk):
      for ch in range(mreal // L, MW // L):
        msg_v[e, pl.ds(ch * L, L)] = zero

    def zcopy(j, _):
      pltpu.sync_copy(zblk, agg_s.at[pl.ds(start + j * 16, 16)])
      return 0
    lax.fori_loop(0, 640 // 16, zcopy, 0)
    if with_cnt:
      pltpu.sync_copy(zcnt, cnt_s.at[pl.ds(start, 640)])
    plsc.subcore_barrier()

    ebase = s * ept if feature_split else (c * NS + s) * ept
    dcb = dc_v[...]

    def issue_idx(k, b):
      base = ebase + k * chunk
      pltpu.make_async_copy(src_hbm.at[pl.ds(base, chunk)], src_c[b],
                            isem[b]).start()
      pltpu.make_async_copy(dst_hbm.at[pl.ds(base, chunk)], dst_c[b],
                            isem[b]).start()

    def wait_idx(k, b):
      base = ebase + k * chunk
      pltpu.make_async_copy(src_hbm.at[pl.ds(base, chunk)], src_c[b],
                            isem[b]).wait()
      pltpu.make_async_copy(dst_hbm.at[pl.ds(base, chunk)], dst_c[b],
                            isem[b]).wait()

    def qcomp(b):
      for g in range(chunk // L):
        sv = src_c[b][pl.ds(g * L, L)]
        dv = dst_c[b][pl.ds(g * L, L)]
        yj = plsc.load_gather(y_v, [sv])
        yi = plsc.load_gather(y_v, [dv])
        q0 = 1.0 / (1.0 + jnp.exp(yi - yj - dcb))
        validf = jnp.where(sv != dv, 1.0, 0.0).astype(jnp.float32)
        q0 = q0 * validf
        q0_v[b][pl.ds(g * L, L)] = q0
        q1_v[b][pl.ds(g * L, L)] = validf - q0
        val_v[b][pl.ds(g * L, L)] = validf

    def issue_gather(b):
      @pl.when(c == 0)
      def _():
        pltpu.make_async_copy(ta_hbm.at[src_c[b]], rows_v[b],
                              gsem[b]).start()
      @pl.when(c == 1)
      def _():
        pltpu.make_async_copy(tb_hbm.at[src_c[b]], rows_v[b],
                              gsem[b]).start()

    def wait_gather(b):
      pltpu.make_async_copy(ta_hbm.at[src_c[b]], rows_v[b], gsem[b]).wait()

    def combine_scatter(b):
      def comb(e, _):
        eix = jnp.zeros((L,), jnp.int32) + e
        q0b = plsc.load_gather(q0_v[b], [eix])
        q1b = plsc.load_gather(q1_v[b], [eix])
        for ch in range(mreal // L):
          r0 = rows_v[b][e, pl.ds(ch * L, L)]
          r1 = rows_v[b][e, pl.ds(mreal + ch * L, L)]
          msg_v[e, pl.ds(ch * L, L)] = q0b * r0 + q1b * r1
        return 0
      lax.fori_loop(0, chunk, comb, 0)
      pltpu.sync_copy(msg_v, agg_s.at[dst_c[b]], add=True)
      if with_cnt:
        @pl.when(c == 0)
        def _():
          pltpu.sync_copy(val_v[b], cnt_s.at[dst_c[b]], add=True)

    n_pipe = nchunks - 2 if nchunks % 2 == 0 else nchunks - 3

    issue_idx(0, 0)
    issue_idx(1, 1)
    wait_idx(0, 0)
    qcomp(0)
    issue_gather(0)

    def pipe_pair(j, _):
      k = 2 * j
      for b in (0, 1):
        kk = k + b
        wait_idx(kk + 1, 1 - b)
        qcomp(1 - b)
        issue_gather(1 - b)
        wait_gather(b)
        combine_scatter(b)
        issue_idx(kk + 2, b)
      return 0

    lax.fori_loop(0, n_pipe // 2, pipe_pair, 0)

    for k in range(n_pipe, nchunks):
      b = k & 1
      if k + 1 < nchunks:
        wait_idx(k + 1, 1 - b)
        qcomp(1 - b)
        issue_gather(1 - b)
      wait_gather(b)
      combine_scatter(b)
      if k + 2 < nchunks:
        issue_idx(k + 2, b)

    plsc.subcore_barrier()

    def wout(j, _):
      pltpu.sync_copy(agg_s.at[pl.ds(start + j * 32, 32)],
                      msg_v.at[pl.ds(0, 32)])
      pltpu.sync_copy(msg_v.at[pl.ds(0, 32)],
                      agg_out.at[c, pl.ds(start + j * 32, 32)])
      return 0
    lax.fori_loop(0, 640 // 32, wout, 0)
    if with_cnt:
      @pl.when(c == 0)
      def _():
        pltpu.sync_copy(cnt_s.at[pl.ds(start, 640)], zcnt)
        pltpu.sync_copy(zcnt, cnt_out.at[pl.ds(start, 640)])

  return pl.kernel(
      body, out_type=out_type, mesh=mesh, scratch_types=scratch,
      compiler_params=pltpu.CompilerParams(needs_layout_passes=False,
                                           use_tc_tiling_on_sc=False))


_sc_layer1 = _sc_edge_kernel(128, feature_split=True, with_cnt=True, chunk=48)
_sc_layer2 = _sc_edge_kernel(64, feature_split=False, with_cnt=False,
                             chunk=80)


_BLK = 1000


def _k1_body(x_ref, w_ref, du_ref, xwa_ref, xwb_ref, ys_ref):
  xb = x_ref[...]
  xw = jnp.dot(xb, w_ref[...], preferred_element_type=jnp.float32)
  xwa_ref[...] = xw[:, :HID]
  xwb_ref[...] = xw[:, HID:]
  ys_ref[...] = jnp.dot(xb, du_ref[...], preferred_element_type=jnp.float32)


def _k1_call(x, w1r, du):
  grid = N // _BLK
  return pl.pallas_call(
      _k1_body,
      grid=(grid,),
      in_specs=[
          pl.BlockSpec((_BLK, IN_CH), lambda i: (i, 0)),
          pl.BlockSpec((IN_CH, 2 * HID), lambda i: (0, 0)),
          pl.BlockSpec((IN_CH, 1), lambda i: (0, 0)),
      ],
      out_specs=[
          pl.BlockSpec((_BLK, HID), lambda i: (i, 0)),
          pl.BlockSpec((_BLK, HID), lambda i: (i, 0)),
          pl.BlockSpec((_BLK, 1), lambda i: (i, 0)),
      ],
      out_shape=[
          jax.ShapeDtypeStruct((N, HID), jnp.float32),
          jax.ShapeDtypeStruct((N, HID), jnp.float32),
          jax.ShapeDtypeStruct((N, 1), jnp.float32),
      ],
  )(x, w1r, du)


def _norm_relu(agg, xw, cnt, brow, qc):
  M = agg.shape[-1]
  self_msg = qc[0, 0] * xw[:, :M] + qc[0, 1] * xw[:, M:]
  return jnp.maximum((agg + self_msg) / (cnt + 1.0) + brow, 0.0)


def _k2_body(agg_ref, xwa_ref, xwb_ref, cnt_ref, b_ref, c1_ref, w2_ref,
             du_ref, xw2_ref, ys2_ref):
  qc = jax.nn.softmax(c1_ref[...], axis=1)
  cnt = cnt_ref[...]
  b = b_ref[...]
  h_lo = _norm_relu(agg_ref[0], xwa_ref[...], cnt, b[0:1, :], qc)
  h_hi = _norm_relu(agg_ref[1], xwb_ref[...], cnt, b[1:2, :], qc)
  hb = jnp.concatenate([h_lo, h_hi], axis=1)
  xw2_ref[...] = jnp.dot(hb, w2_ref[...], preferred_element_type=jnp.float32)
  ys2_ref[...] = jnp.dot(hb, du_ref[...], preferred_element_type=jnp.float32)


def _k2_call(agg1, xwa, xwb, cnt, b1, c1, w2, du2):
  grid = N // _BLK
  return pl.pallas_call(
      _k2_body,
      grid=(grid,),
      in_specs=[
          pl.BlockSpec((NC, _BLK, HID // 2), lambda i: (0, i, 0)),
          pl.BlockSpec((_BLK, HID), lambda i: (i, 0)),
          pl.BlockSpec((_BLK, HID), lambda i: (i, 0)),
          pl.BlockSpec((_BLK, 1), lambda i: (i, 0)),
          pl.BlockSpec((2, HID // 2), lambda i: (0, 0)),
          pl.BlockSpec((1, 2), lambda i: (0, 0)),
          pl.BlockSpec((HID, 2 * OUT2), lambda i: (0, 0)),
          pl.BlockSpec((HID, 1), lambda i: (0, 0)),
      ],
      out_specs=[
          pl.BlockSpec((_BLK, 2 * OUT2), lambda i: (i, 0)),
          pl.BlockSpec((_BLK, 1), lambda i: (i, 0)),
      ],
      out_shape=[
          jax.ShapeDtypeStruct((N, 2 * OUT2), jnp.float32),
          jax.ShapeDtypeStruct((N, 1), jnp.float32),
      ],
  )(agg1, xwa, xwb, cnt, b1, c1, w2, du2)


def _k3_body(agg2_ref, xw2_ref, cnt_ref, b_ref, c2_ref, batch_ref, wfc_ref,
             bfc_ref, out_ref):
  qc = jax.nn.softmax(c2_ref[...], axis=1)
  cnt = cnt_ref[...]
  xw2 = xw2_ref[...]
  agg2 = agg2_ref[0][:, :OUT2] + agg2_ref[1][:, :OUT2]
  self_msg = qc[0, 0] * xw2[:, :OUT2] + qc[0, 1] * xw2[:, OUT2:]
  h2 = jnp.maximum((agg2 + self_msg) / (cnt + 1.0) + b_ref[...], 0.0)
  gids = lax.broadcasted_iota(jnp.int32, (1, NUM_GRAPHS), 1)
  seg = (batch_ref[...] == gids).astype(jnp.float32)
  gsum = lax.dot_general(seg, h2, (((0,), (0,)), ((), ())),
                         preferred_element_type=jnp.float32)
  counts = lax.dot_general(seg, jnp.ones((N, 1), jnp.float32),
                           (((0,), (0,)), ((), ())),
                           preferred_element_type=jnp.float32)
  g = gsum / jnp.maximum(counts, 1.0)
  out_ref[...] = jnp.dot(g, wfc_ref[...],
                         preferred_element_type=jnp.float32) + bfc_ref[...]


def _k3_call(agg2, xw2, cnt, b2, c2, batch2d, wfc, bfc):
  return pl.pallas_call(
      _k3_body,
      out_shape=jax.ShapeDtypeStruct((NUM_GRAPHS, NUM_CLASSES), jnp.float32),
  )(agg2, xw2, cnt, b2, c2, batch2d, wfc, bfc)


def kernel(x, edge_index, batch, W1, u1, c1, b1, W2, u2, c2, b2, Wfc, bfc):
  pad = jnp.zeros((E_PAD - E,), jnp.int32)
  src = jnp.concatenate([edge_index[0], pad])
  dst = jnp.concatenate([edge_index[1], pad])
  H2 = HID // 2
  W1r = jnp.concatenate([W1[:, 0:H2], W1[:, HID:HID + H2],
                         W1[:, H2:HID], W1[:, HID + H2:]], axis=1)
  du1 = (u1[:, 0] - u1[:, 1])[:, None]
  dc1 = jnp.full((L,), c1[0] - c1[1], jnp.float32)
  du2 = (u2[:, 0] - u2[:, 1])[:, None]
  dc2 = jnp.full((L,), c2[0] - c2[1], jnp.float32)

  xwa, xwb, ys1 = _k1_call(x, W1r, du1)
  agg1, cnt = _sc_layer1(xwa, xwb, jnp.ravel(ys1), dc1, src, dst)

  xw2, ys2 = _k2_call(agg1, xwa, xwb, cnt.reshape(N, 1), b1.reshape(2, H2),
                      c1.reshape(1, 2), W2, du2)
  agg2 = _sc_layer2(xw2, xw2, jnp.ravel(ys2), dc2, src, dst)

  return _k3_call(agg2, xw2, cnt.reshape(N, 1), b2.reshape(1, OUT2),
                  c2.reshape(1, 2), batch.reshape(N, 1), Wfc, bfc[None, :])
